# baseline (speedup 1.0000x reference)
import functools

import jax
import jax.numpy as jnp
from jax.experimental import pallas as pl
from jax.experimental.pallas import tpu as pltpu

SEQ_LEN = 4096
BSZ = 2
EMBED_DIM = 4096
N_EXPERTS = 31
TOP_K = 3
ALPHA = 0.001

E_PAD = 32
RB = 512
N_ROWS = SEQ_LEN * BSZ
N_BLOCKS = N_ROWS // RB
HALF_BLOCKS = (N_ROWS // 2) // RB
NEG = -1e30


def _copy(x_hbm, x_buf, sem, step, slot):
    x2d = x_hbm.reshape(N_ROWS, EMBED_DIM)
    return pltpu.make_async_copy(
        x2d.at[pl.ds(step * RB, RB), :],
        x_buf.at[slot],
        sem.at[slot],
    )


def _gate_kernel(x_hbm, w_ref, idx_ref, wgt_ref, aux_ref, x_buf, sem,
                 cnt_acc, sum_acc):
    i = pl.program_id(0)
    slot = jax.lax.rem(i, 2)

    @pl.when(i == 0)
    def _init():
        cnt_acc[...] = jnp.zeros_like(cnt_acc)
        sum_acc[...] = jnp.zeros_like(sum_acc)
        _copy(x_hbm, x_buf, sem, 0, 0).start()

    @pl.when(i + 1 < N_BLOCKS)
    def _prefetch():
        _copy(x_hbm, x_buf, sem, i + 1, 1 - slot).start()

    _copy(x_hbm, x_buf, sem, i, slot).wait()

    logits = jax.lax.dot_general(
        w_ref[...], x_buf[slot],
        dimension_numbers=(((1,), (1,)), ((), ())),
        preferred_element_type=jnp.float32,
        precision=jax.lax.Precision.DEFAULT,
    )
    sub = jax.lax.broadcasted_iota(jnp.int32, (E_PAD, RB), 0)
    logits = jnp.where(sub < N_EXPERTS, logits, NEG)

    m = jnp.max(logits, axis=0, keepdims=True)
    p = jnp.exp(logits - m)
    z = jnp.sum(p, axis=0, keepdims=True)

    cur = logits
    vals = []
    idxs = []
    for _ in range(TOP_K):
        v = jnp.max(cur, axis=0, keepdims=True)
        hit = cur >= v
        ix = jnp.min(jnp.where(hit, sub, E_PAD), axis=0, keepdims=True)
        vals.append(v)
        idxs.append(ix)
        cur = jnp.where(sub == ix, NEG, cur)

    ts = [jnp.exp(v - m) / z for v in vals]
    denom = ts[0] + ts[1] + ts[2] + 1e-20
    idx_ref[...] = jnp.concatenate(idxs, axis=0)
    wgt_ref[...] = jnp.concatenate([t / denom for t in ts], axis=0)

    h = i // HALF_BLOCKS
    onehot_h = (jax.lax.broadcasted_iota(jnp.int32, (1, 2), 1) == h).astype(jnp.float32)

    scores_sum = jnp.sum(p / z, axis=1, keepdims=True)
    sum_acc[...] += scores_sum * onehot_h
    cnt = jnp.zeros((E_PAD, 1), dtype=jnp.float32)
    for k in range(TOP_K):
        cnt += jnp.sum((sub == idxs[k]).astype(jnp.float32), axis=1, keepdims=True)
    cnt_acc[...] += cnt * onehot_h

    @pl.when(i == N_BLOCKS - 1)
    def _finalize():
        scale = ALPHA * (1.0 / BSZ) * N_EXPERTS / (SEQ_LEN * SEQ_LEN * TOP_K)
        aux_ref[...] = (jnp.sum(cnt_acc[...] * sum_acc[...]) * scale).reshape(1, 1)


@functools.partial(jax.jit, static_argnums=())
def _gate(x, w_pad):
    idxT, wgtT, aux = pl.pallas_call(
        _gate_kernel,
        grid=(N_BLOCKS,),
        in_specs=[
            pl.BlockSpec(memory_space=pl.ANY),
            pl.BlockSpec((E_PAD, EMBED_DIM), lambda i: (0, 0)),
        ],
        out_specs=[
            pl.BlockSpec((TOP_K, RB), lambda i: (0, i)),
            pl.BlockSpec((TOP_K, RB), lambda i: (0, i)),
            pl.BlockSpec((1, 1), lambda i: (0, 0)),
        ],
        out_shape=[
            jax.ShapeDtypeStruct((TOP_K, N_ROWS), jnp.int32),
            jax.ShapeDtypeStruct((TOP_K, N_ROWS), jnp.float32),
            jax.ShapeDtypeStruct((1, 1), jnp.float32),
        ],
        scratch_shapes=[
            pltpu.VMEM((2, RB, EMBED_DIM), jnp.float32),
            pltpu.SemaphoreType.DMA((2,)),
            pltpu.VMEM((E_PAD, 2), jnp.float32),
            pltpu.VMEM((E_PAD, 2), jnp.float32),
        ],
    )(x, w_pad)
    return idxT.T, wgtT.T, aux[0, 0]


def kernel(x, weight):
    w_pad = jnp.zeros((E_PAD, EMBED_DIM), dtype=weight.dtype).at[:N_EXPERTS].set(weight)
    return _gate(x, w_pad)
